# baseline (speedup 1.0000x reference)
import functools

import jax
import jax.numpy as jnp
from jax import lax
from jax.experimental import pallas as pl
from jax.experimental.pallas import tpu as pltpu
from jax.experimental.pallas import tpu_sc as plsc

_N = 32768
_DIM = 768
_NC = 2
_NS = 16
_NW = _NC * _NS
_RPW = _N // _NW
_REP = 64
_CHW = _REP * _DIM
_NDMA = _RPW // _REP
_VJ = _DIM // 16

_mesh = plsc.VectorSubcoreMesh(core_axis_name="c", subcore_axis_name="s")


@functools.partial(
    pl.kernel,
    out_type=(
        jax.ShapeDtypeStruct((_N * _DIM,), jnp.float32),
        jax.ShapeDtypeStruct((_N,), jnp.float32),
        jax.ShapeDtypeStruct((_N,), jnp.int32),
    ),
    mesh=_mesh,
    scratch_types=[
        pltpu.VMEM((_DIM,), jnp.float32),
        pltpu.VMEM((_CHW,), jnp.float32),
        pltpu.VMEM((_RPW,), jnp.float32),
        pltpu.VMEM((_RPW,), jnp.int32),
        pltpu.SemaphoreType.DMA,
    ],
)
def _sc_fill(x0_hbm, sel_hbm, sc_hbm, idx_hbm, vrow, vbuf, vsc, vidx, sem):
    wid = lax.axis_index("s") * _NC + lax.axis_index("c")
    pltpu.sync_copy(x0_hbm, vrow)

    ones16 = jnp.ones((16,), jnp.float32)
    zeros16 = jnp.zeros((16,), jnp.int32)

    def fill_row(r, carry):
        for j in range(_VJ):
            vbuf[pl.ds(r * _DIM + j * 16, 16)] = vrow[pl.ds(j * 16, 16)]
        return carry

    lax.fori_loop(0, _REP, fill_row, 0)

    def fill_const(r, carry):
        vsc[pl.ds(r * 16, 16)] = ones16
        vidx[pl.ds(r * 16, 16)] = zeros16
        return carry

    lax.fori_loop(0, _RPW // 16, fill_const, 0)

    base = wid * (_RPW * _DIM)
    cps = []
    for d in range(_NDMA):
        c = pltpu.make_async_copy(
            vbuf, sel_hbm.at[pl.ds(base + d * _CHW, _CHW)], sem)
        c.start()
        cps.append(c)
    pltpu.sync_copy(vsc, sc_hbm.at[pl.ds(wid * _RPW, _RPW)])
    pltpu.sync_copy(vidx, idx_hbm.at[pl.ds(wid * _RPW, _RPW)])
    for c in cps:
        c.wait()


def kernel(x, W1, b1, W2, b2):
    x0 = x[0].reshape(_DIM)
    sel, sc, idx = _sc_fill(x0)
    return sel.reshape(_N, 1, _DIM), sc.reshape(_N, 1), idx.reshape(_N, 1)


# speedup vs baseline: 1.9205x; 1.9205x over previous
import jax
import jax.numpy as jnp
from jax.experimental import pallas as pl

_N = 32768
_DIM = 768
_BN = 2048


def _fill_kernel(x_ref, sel_ref, sc_ref, idx_ref):
    sel_ref[...] = jnp.broadcast_to(
        x_ref[0:1, :].reshape(1, 1, _DIM), sel_ref.shape)
    sc_ref[...] = jnp.ones_like(sc_ref)
    idx_ref[...] = jnp.zeros_like(idx_ref)


def kernel(x, W1, b1, W2, b2):
    sel, sc, idx = pl.pallas_call(
        _fill_kernel,
        grid=(_N // _BN,),
        in_specs=[pl.BlockSpec((8, _DIM), lambda i: (0, 0))],
        out_specs=[
            pl.BlockSpec((_BN, 1, _DIM), lambda i: (i, 0, 0)),
            pl.BlockSpec((_BN,), lambda i: (i,)),
            pl.BlockSpec((_BN,), lambda i: (i,)),
        ],
        out_shape=[
            jax.ShapeDtypeStruct((_N, 1, _DIM), jnp.float32),
            jax.ShapeDtypeStruct((_N,), jnp.float32),
            jax.ShapeDtypeStruct((_N,), jnp.int32),
        ],
    )(x)
    return (sel, sc.reshape(_N, 1), idx.reshape(_N, 1))
